# baseline (speedup 1.0000x reference)
import functools

import jax
import jax.numpy as jnp
from jax import lax
from jax.experimental import pallas as pl
from jax.experimental.pallas import tpu as pltpu
from jax.experimental.pallas import tpu_sc as plsc

VOCAB = 1000000
D = 32
B = 4096
S = 50
L = 16
NC, NS = 2, 16
NW = NC * NS
ROWS_W = B // NW
R = 16
NP = ROWS_W // R
KW = (S + L - 1) // L
QROW = 4 * D


def _body(desc_t, word4_hbm, weight_hbm, out_hbm,
          idx_v, qid_v, cb_v, emb_v, w_v, wexp_v, out_v, gsem, wsem):
    wid = lax.axis_index("s") * NC + lax.axis_index("c")
    iota = lax.iota(jnp.int32, L)
    col0 = wid * ROWS_W

    pltpu.sync_copy(desc_t.at[:, pl.ds(col0, ROWS_W)], idx_v)

    def pre_body(t, _):
        j = t >> 3
        c = (t & 7) * L
        v = idx_v[j, pl.ds(c, L)]
        qid_v[j, pl.ds(c, L)] = lax.shift_right_logical(v, 2)
        cb_v[j, pl.ds(c, L)] = lax.shift_left(jnp.bitwise_and(v, 3), 5)
        return _

    lax.fori_loop(0, S * ROWS_W // L, pre_body, 0)

    def do_pass(p, _):
        def fire(j, _):
            pltpu.async_copy(
                word4_hbm.at[qid_v.at[j, pl.ds(p * R, R)]],
                emb_v.at[pl.ds(j * R, R), :], gsem)
            pltpu.async_copy(
                weight_hbm.at[idx_v.at[j, pl.ds(p * R, R)]],
                w_v.at[pl.ds(j * R, R)], wsem)
            return _

        lax.fori_loop(0, S, fire, 0)

        def drain(j, _):
            pltpu.make_async_copy(
                word4_hbm.at[qid_v.at[0, pl.ds(0, R)]],
                emb_v.at[pl.ds(0, R), :], gsem).wait()
            pltpu.make_async_copy(
                weight_hbm.at[idx_v.at[0, pl.ds(0, R)]],
                w_v.at[pl.ds(0, R)], wsem).wait()
            return _

        lax.fori_loop(0, S, drain, 0)

        def row_body(rr, _):
            col = p * R + rr
            wvecs = []
            for k in range(KW):
                idxs = jnp.minimum(k * L + iota, S - 1) * R + rr
                wvecs.append(plsc.load_gather(w_v, [idxs]))
            masks = [(k * L + iota) < S for k in range(KW)]
            mvec = jnp.where(masks[0], wvecs[0], -1e30)
            for k in range(1, KW):
                mvec = jnp.maximum(mvec, jnp.where(masks[k], wvecs[k], -1e30))
            mx = jnp.max(mvec)
            svec = jnp.zeros((L,), jnp.float32)
            evecs = []
            for k in range(KW):
                e_k = jnp.where(masks[k], jnp.exp(wvecs[k] - mx), 0.0)
                evecs.append(e_k)
                svec = svec + e_k
            inv = jnp.ones((L,), jnp.float32) / lax.broadcast(jnp.sum(svec), (L,))
            wbase = rr * (KW * L)
            for k in range(KW):
                wexp_v[pl.ds(wbase + k * L, L)] = evecs[k] * inv
            acc0 = jnp.zeros((L,), jnp.float32)
            acc1 = jnp.zeros((L,), jnp.float32)
            for j in range(S):
                wb = plsc.load_gather(wexp_v, [lax.broadcast(wbase + j, (L,))])
                cb = plsc.load_gather(
                    cb_v, [jnp.full((L,), j, jnp.int32), lax.broadcast(col, (L,))])
                rowv = lax.broadcast(j * R + rr, (L,))
                acc0 = acc0 + wb * plsc.load_gather(emb_v, [rowv, cb + iota])
                acc1 = acc1 + wb * plsc.load_gather(emb_v, [rowv, cb + iota + L])
            out_v[rr, pl.ds(0, L)] = acc0
            out_v[rr, pl.ds(L, L)] = acc1
            return _

        lax.fori_loop(0, R, row_body, 0)

        pltpu.sync_copy(out_v, out_hbm.at[pl.ds(col0 + p * R, R), :])
        return _

    lax.fori_loop(0, NP, do_pass, 0)


@jax.jit
def _run(desc_t, word4, weight_flat):
    mesh = plsc.VectorSubcoreMesh(core_axis_name="c", subcore_axis_name="s")
    return pl.kernel(
        _body,
        out_type=jax.ShapeDtypeStruct((B, D), jnp.float32),
        mesh=mesh,
        scratch_types=[
            pltpu.VMEM((S, ROWS_W), jnp.int32),
            pltpu.VMEM((S, ROWS_W), jnp.int32),
            pltpu.VMEM((S, ROWS_W), jnp.int32),
            pltpu.VMEM((S * R, QROW), jnp.float32),
            pltpu.VMEM((S * R,), jnp.float32),
            pltpu.VMEM((R * KW * L,), jnp.float32),
            pltpu.VMEM((R, D), jnp.float32),
            pltpu.SemaphoreType.DMA,
            pltpu.SemaphoreType.DMA,
        ],
        compiler_params=pltpu.CompilerParams(
            needs_layout_passes=False, use_tc_tiling_on_sc=True),
    )(desc_t, word4, weight_flat)


def kernel(desc, word_table, weight_table):
    return _run(desc.T, word_table.reshape(VOCAB // 4, QROW),
                weight_table.reshape(VOCAB))
